# Initial kernel scaffold; baseline (speedup 1.0000x reference)
import jax
import jax.numpy as jnp
from jax.experimental import pallas as pl


def kernel(x, Wg, bg, W1, b1, W2, b2, W3, b3):
    raise NotImplementedError("write your pallas kernel here")


# speedup vs baseline: 5.2873x; 5.2873x over previous
import functools

import jax
import jax.numpy as jnp
from jax.experimental import pallas as pl
from jax.experimental.pallas import tpu as pltpu


def _route_body(x_ref, wg_ref, tok_ref, prob_ref, xg_ref):
    B, S, DIN = x_ref.shape
    E = wg_ref.shape[1]
    lts = []
    for b in range(B):
        lt = jax.lax.dot_general(
            wg_ref[...], x_ref[b],
            (((0,), (1,)), ((), ())),
            preferred_element_type=jnp.float32,
        )
        lts.append(lt)
    lt = jnp.concatenate(lts, axis=0)

    iot = jax.lax.broadcasted_iota(jnp.int32, lt.shape, 1)
    neg = jnp.float32(-jnp.inf)

    m1 = jnp.max(lt, axis=1)
    i1 = jnp.min(jnp.where(lt == m1[:, None], iot, S), axis=1)
    ltm = jnp.where(iot == i1[:, None], neg, lt)
    m2 = jnp.max(ltm, axis=1)
    i2 = jnp.min(jnp.where(ltm == m2[:, None], iot, S), axis=1)

    z = jnp.sum(jnp.exp(lt - m1[:, None]), axis=1)
    p1 = 1.0 / z
    p2 = jnp.exp(m2 - m1) / z

    tok_ref[...] = jnp.stack([i1, i2], axis=1).astype(jnp.int32)
    prob_ref[...] = jnp.stack([p1, p2], axis=1)

    for e in range(E):
        for b in range(B):
            p_row = b * E + e
            for k, ivec in enumerate((i1, i2)):
                t = ivec[p_row]
                xg_ref[e, b * 2 + k : b * 2 + k + 1, :] = x_ref[b, pl.ds(t, 1), :]


def _expert_body(tok_ref, prob_ref, xg_ref, w1_ref, b1_ref, w2_ref, b2_ref,
                 w3_ref, b3_ref, out_ref):
    e = pl.program_id(0)
    E = pl.num_programs(0)
    B = out_ref.shape[0]
    K = xg_ref.shape[1] // B

    @pl.when(e == 0)
    def _():
        out_ref[...] = jnp.zeros_like(out_ref)

    xe = xg_ref[0]
    h = jnp.dot(xe, w1_ref[0], preferred_element_type=jnp.float32)
    h = jnp.maximum(h + b1_ref[pl.ds(e, 1), :], 0.0)
    h = jnp.dot(h, w2_ref[0], preferred_element_type=jnp.float32)
    h = jnp.maximum(h + b2_ref[pl.ds(e, 1), :], 0.0)
    y = jnp.dot(h, w3_ref[0], preferred_element_type=jnp.float32)
    y = y + b3_ref[pl.ds(e, 1), :]

    for b in range(B):
        for k in range(K):
            row = b * K + k
            t = tok_ref[b * E + e, k]
            pr = prob_ref[b * E + e, k]
            out_ref[b, pl.ds(t, 1), :] = (
                out_ref[b, pl.ds(t, 1), :] + pr * y[row : row + 1, :]
            )


@jax.jit
def kernel(x, Wg, bg, W1, b1, W2, b2, W3, b3):
    del bg
    B, S, DIN = x.shape
    E = Wg.shape[1]
    DOUT = W1.shape[2]
    K = 2

    tok, prob, xg = pl.pallas_call(
        _route_body,
        out_shape=(
            jax.ShapeDtypeStruct((B * E, K), jnp.int32),
            jax.ShapeDtypeStruct((B * E, K), jnp.float32),
            jax.ShapeDtypeStruct((E, B * K, DIN), jnp.float32),
        ),
    )(x, Wg)

    grid_spec = pltpu.PrefetchScalarGridSpec(
        num_scalar_prefetch=2,
        grid=(E,),
        in_specs=[
            pl.BlockSpec((1, B * K, DIN), lambda e, *_: (e, 0, 0)),
            pl.BlockSpec((1, DIN, DOUT), lambda e, *_: (e, 0, 0)),
            pl.BlockSpec((E, 1), lambda e, *_: (0, 0)),
            pl.BlockSpec((1, DOUT, DOUT), lambda e, *_: (e, 0, 0)),
            pl.BlockSpec((E, 1), lambda e, *_: (0, 0)),
            pl.BlockSpec((1, DOUT, DOUT), lambda e, *_: (e, 0, 0)),
            pl.BlockSpec((E, 1), lambda e, *_: (0, 0)),
        ],
        out_specs=pl.BlockSpec((B, S, DOUT), lambda e, *_: (0, 0, 0)),
    )
    out = pl.pallas_call(
        _expert_body,
        grid_spec=grid_spec,
        out_shape=jax.ShapeDtypeStruct((B, S, DOUT), jnp.float32),
        compiler_params=pltpu.CompilerParams(
            dimension_semantics=("arbitrary",),
        ),
    )(tok, prob, xg, W1, b1, W2, b2, W3, b3)
    return out


# speedup vs baseline: 5.5738x; 1.0542x over previous
import jax
import jax.numpy as jnp
from jax.experimental import pallas as pl
from jax.experimental.pallas import tpu as pltpu

_K = 2


def _moe_body(x_ref, wg_ref, w1_ref, b1_ref, w2_ref, b2_ref, w3_ref, b3_ref,
              out_ref, tok_s, prob_s):
    e = pl.program_id(0)
    E = pl.num_programs(0)
    B, S, DIN = x_ref.shape

    @pl.when(e == 0)
    def _():
        out_ref[...] = jnp.zeros_like(out_ref)

        lts = []
        for b in range(B):
            lt = jax.lax.dot_general(
                wg_ref[...], x_ref[b],
                (((0,), (1,)), ((), ())),
                preferred_element_type=jnp.float32,
            )
            lts.append(lt)
        lt = jnp.concatenate(lts, axis=0)

        iot = jax.lax.broadcasted_iota(jnp.int32, lt.shape, 1)
        neg = jnp.float32(-jnp.inf)
        m1 = jnp.max(lt, axis=1)
        i1 = jnp.min(jnp.where(lt == m1[:, None], iot, S), axis=1)
        ltm = jnp.where(iot == i1[:, None], neg, lt)
        m2 = jnp.max(ltm, axis=1)
        i2 = jnp.min(jnp.where(ltm == m2[:, None], iot, S), axis=1)
        z = jnp.sum(jnp.exp(lt - m1[:, None]), axis=1)
        p1 = 1.0 / z
        p2 = jnp.exp(m2 - m1) / z

        for p in range(B * E):
            tok_s[p, 0] = i1[p]
            tok_s[p, 1] = i2[p]
            prob_s[p, 0] = p1[p]
            prob_s[p, 1] = p2[p]

    rows = []
    for b in range(B):
        for k in range(_K):
            t = tok_s[b * E + e, k]
            rows.append(x_ref[b, pl.ds(t, 1), :])
    xe = jnp.concatenate(rows, axis=0)

    h = jnp.dot(xe, w1_ref[0], preferred_element_type=jnp.float32)
    h = jnp.maximum(h + b1_ref[pl.ds(e, 1), :], 0.0)
    h = jnp.dot(h, w2_ref[0], preferred_element_type=jnp.float32)
    h = jnp.maximum(h + b2_ref[pl.ds(e, 1), :], 0.0)
    y = jnp.dot(h, w3_ref[0], preferred_element_type=jnp.float32)
    y = y + b3_ref[pl.ds(e, 1), :]

    for b in range(B):
        for k in range(_K):
            row = b * _K + k
            t = tok_s[b * E + e, k]
            pr = prob_s[b * E + e, k]
            out_ref[b, pl.ds(t, 1), :] = (
                out_ref[b, pl.ds(t, 1), :] + pr * y[row : row + 1, :]
            )


@jax.jit
def kernel(x, Wg, bg, W1, b1, W2, b2, W3, b3):
    del bg
    B, S, DIN = x.shape
    E = Wg.shape[1]
    DOUT = W1.shape[2]

    return pl.pallas_call(
        _moe_body,
        grid=(E,),
        in_specs=[
            pl.BlockSpec((B, S, DIN), lambda e: (0, 0, 0)),
            pl.BlockSpec((DIN, E), lambda e: (0, 0)),
            pl.BlockSpec((1, DIN, DOUT), lambda e: (e, 0, 0)),
            pl.BlockSpec((E, 1), lambda e: (0, 0)),
            pl.BlockSpec((1, DOUT, DOUT), lambda e: (e, 0, 0)),
            pl.BlockSpec((E, 1), lambda e: (0, 0)),
            pl.BlockSpec((1, DOUT, DOUT), lambda e: (e, 0, 0)),
            pl.BlockSpec((E, 1), lambda e: (0, 0)),
        ],
        out_specs=pl.BlockSpec((B, S, DOUT), lambda e: (0, 0, 0)),
        out_shape=jax.ShapeDtypeStruct((B, S, DOUT), jnp.float32),
        scratch_shapes=[
            pltpu.SMEM((B * E, _K), jnp.int32),
            pltpu.SMEM((B * E, _K), jnp.float32),
        ],
        compiler_params=pltpu.CompilerParams(
            dimension_semantics=("arbitrary",),
            vmem_limit_bytes=100 * 1024 * 1024,
        ),
    )(x, Wg, W1, b1, W2, b2, W3, b3)
